# baseline (speedup 1.0000x reference)
import functools

import jax
import jax.numpy as jnp
from jax import lax
from jax.experimental import pallas as pl
from jax.experimental.pallas import tpu as pltpu
from jax.experimental.pallas import tpu_sc as plsc

VOCAB = 1000000
EMBED = 64
BATCH = 16384

L = 16
NC = 2
NS = 16
NW = NC * NS
BPW = BATCH // NW
NG = BPW // L

_LN2 = 0.6931471805599453
_SQRT2 = 1.4142135623730951


def _vlog(x):
    bits = lax.bitcast_convert_type(x, jnp.int32)
    e = lax.shift_right_arithmetic(bits, 23) - 127
    m = lax.bitcast_convert_type(
        (bits & 0x7FFFFF) | 0x3F800000, jnp.float32)
    big = m >= _SQRT2
    m = jnp.where(big, m * 0.5, m)
    e = (e + jnp.where(big, 1, 0)).astype(jnp.float32)
    t = (m - 1.0) / (m + 1.0)
    t2 = t * t
    p = 1.0 + t2 * (1 / 3 + t2 * (1 / 5 + t2 * (1 / 7 + t2 * (1 / 9))))
    lnm = 2.0 * t * p
    return jnp.where(x <= 0.0, -jnp.inf, e * _LN2 + lnm)


def _tec_body(i_hbm, k_hbm, x_hbm, w_hbm, W_hbm, bu_hbm, bv_hbm, out_hbm,
              idx_i, idx_k, wi_v, wk_v, bu_v, bv_v, xv, wv, accv, tr_v,
              s1, s2, s3, s4):
    wid = lax.axis_index("s") * NC + lax.axis_index("c")
    base = pl.multiple_of(wid * BPW, BPW)

    pltpu.sync_copy(i_hbm.at[pl.ds(base, BPW)], idx_i)
    pltpu.sync_copy(k_hbm.at[pl.ds(base, BPW)], idx_k)
    c1 = pltpu.async_copy(W_hbm.at[idx_i], wi_v, s1)
    c2 = pltpu.async_copy(W_hbm.at[idx_k], wk_v, s2)
    c3 = pltpu.async_copy(bu_hbm.at[idx_i], bu_v, s3)
    c4 = pltpu.async_copy(bv_hbm.at[idx_k], bv_v, s4)
    pltpu.sync_copy(x_hbm.at[pl.ds(base, BPW)], xv)
    pltpu.sync_copy(w_hbm.at[pl.ds(base, BPW)], wv)
    c1.wait()
    c2.wait()
    c3.wait()
    c4.wait()

    lane = lax.iota(jnp.int32, L)

    def group(g, acc):
        gb = pl.multiple_of(g * L, L)
        for b in range(L):
            row = gb + b
            pr = None
            for c in range(EMBED // L):
                a_ = wi_v[row, pl.ds(c * L, L)]
                b_ = wk_v[row, pl.ds(c * L, L)]
                pr = a_ * b_ if pr is None else pr + a_ * b_
            plsc.store_scatter(tr_v, [lane * L + b], pr)
        sim = jnp.zeros((L,), jnp.float32)
        for j in range(L):
            sim = sim + tr_v[pl.ds(j * L, L)]
        bu16 = bu_v[pl.ds(gb, L)]
        bv16 = bv_v[pl.ds(gb, L)]
        x16 = xv[pl.ds(gb, L)]
        w16 = wv[pl.ds(gb, L)]
        r = sim + bu16 + bv16 - _vlog(x16)
        return acc + r * r * w16 * 0.5

    acc = lax.fori_loop(0, NG, group, jnp.zeros((L,), jnp.float32))
    accv[...] = acc
    pltpu.sync_copy(accv, out_hbm.at[wid])


@jax.jit
def _glove_sc(i, k, x_ik, w, W, bu, bv):
    mesh = plsc.VectorSubcoreMesh(core_axis_name="c", subcore_axis_name="s")
    f = functools.partial(
        pl.kernel,
        mesh=mesh,
        compiler_params=pltpu.CompilerParams(
            needs_layout_passes=False, use_tc_tiling_on_sc=False),
        out_type=jax.ShapeDtypeStruct((NW, L), jnp.float32),
        scratch_types=[
            pltpu.VMEM((BPW,), jnp.int32),
            pltpu.VMEM((BPW,), jnp.int32),
            pltpu.VMEM((BPW, EMBED), jnp.float32),
            pltpu.VMEM((BPW, EMBED), jnp.float32),
            pltpu.VMEM((BPW,), jnp.float32),
            pltpu.VMEM((BPW,), jnp.float32),
            pltpu.VMEM((BPW,), jnp.float32),
            pltpu.VMEM((BPW,), jnp.float32),
            pltpu.VMEM((L,), jnp.float32),
            pltpu.VMEM((L * L,), jnp.float32),
            pltpu.SemaphoreType.DMA,
            pltpu.SemaphoreType.DMA,
            pltpu.SemaphoreType.DMA,
            pltpu.SemaphoreType.DMA,
        ],
    )(_tec_body)
    return f(i, k, x_ik, w, W, bu, bv)


def kernel(i, k, x_ik, w, W, B_v, B_u):
    partials = _glove_sc(i, k, x_ik, w, W,
                         B_u.reshape(VOCAB), B_v.reshape(VOCAB))
    return jnp.sum(partials)
